# Initial kernel scaffold; baseline (speedup 1.0000x reference)
import jax
import jax.numpy as jnp
from jax.experimental import pallas as pl


def kernel(x, edge_index, W_mu, b_mu, W_logstd, b_logstd):
    raise NotImplementedError("write your pallas kernel here")


# speedup vs baseline: 20.9901x; 20.9901x over previous
import functools

import jax
import jax.numpy as jnp
from jax import lax
from jax.experimental import pallas as pl
from jax.experimental.pallas import tpu as pltpu
from jax.experimental.pallas import tpu_sc as plsc

N = 10000
E = 160000
D = 256
H = 128
NC = 2
NS = 16
B = 80
ET = E // NS
NB = ET // B
ZB = 640
R = 1000

_mesh = plsc.VectorSubcoreMesh(core_axis_name="c", subcore_axis_name="s")


@functools.partial(
    pl.kernel,
    out_type=jax.ShapeDtypeStruct((N,), jnp.float32),
    mesh=_mesh,
    scratch_types=[
        pltpu.VMEM((NB, B), jnp.int32),
        pltpu.VMEM((B,), jnp.float32),
        pltpu.VMEM((ZB,), jnp.float32),
        pltpu.VMEM_SHARED((N,), jnp.float32),
    ],
)
def _deg_kernel(dstr, ones_hbm, out, dst_v, ones_v, slab_v, dacc):
    c = lax.axis_index("c")
    s = lax.axis_index("s")

    @pl.when(c == 0)
    def _():
        r0 = jnp.where(s < NS - 1, s * ZB, N - ZB)
        for i in range(ZB // 16):
            slab_v[pl.ds(16 * i, 16)] = jnp.zeros((16,), jnp.float32)
        pltpu.sync_copy(slab_v, dacc.at[pl.ds(r0, ZB)])
        pltpu.sync_copy(ones_hbm, ones_v)
        pltpu.sync_copy(dstr.at[s], dst_v)
        plsc.subcore_barrier()

        def body(j, carry):
            pltpu.sync_copy(ones_v, dacc.at[dst_v.at[j]], add=True)
            return carry

        lax.fori_loop(0, NB, body, 0)
        plsc.subcore_barrier()
        pltpu.sync_copy(dacc.at[pl.ds(r0, ZB)], slab_v)
        pltpu.sync_copy(slab_v, out.at[pl.ds(r0, ZB)])


def _scale_body(deg_ref, x_ref, xsa_ref, xsb_ref, db_ref):
    dinv = lax.rsqrt(deg_ref[...] + 1.0)
    xs = x_ref[...] * dinv
    xsa_ref[...] = xs[:, :H]
    xsb_ref[...] = xs[:, H:]
    db_ref[...] = jnp.broadcast_to(dinv, (R, H))


_scale = pl.pallas_call(
    _scale_body,
    grid=(N // R,),
    in_specs=[
        pl.BlockSpec((R, 1), lambda i: (i, 0)),
        pl.BlockSpec((R, D), lambda i: (i, 0)),
    ],
    out_specs=[
        pl.BlockSpec((R, H), lambda i: (i, 0)),
        pl.BlockSpec((R, H), lambda i: (i, 0)),
        pl.BlockSpec((R, H), lambda i: (i, 0)),
    ],
    out_shape=[
        jax.ShapeDtypeStruct((N, H), jnp.float32),
        jax.ShapeDtypeStruct((N, H), jnp.float32),
        jax.ShapeDtypeStruct((N, H), jnp.float32),
    ],
)


@functools.partial(
    pl.kernel,
    out_type=(
        jax.ShapeDtypeStruct((N, H), jnp.float32),
        jax.ShapeDtypeStruct((N, H), jnp.float32),
    ),
    mesh=_mesh,
    scratch_types=[
        pltpu.VMEM((NB, B), jnp.int32),
        pltpu.VMEM((NB, B), jnp.int32),
        pltpu.VMEM((B, H), jnp.float32),
        pltpu.VMEM_SHARED((N, H), jnp.float32),
        pltpu.SemaphoreType.DMA,
    ],
)
def _agg_kernel(xsa, xsb, srcr, dstr, outa, outb, src_v, dst_v, rows_v, acc, sem):
    c = lax.axis_index("c")
    s = lax.axis_index("s")
    pltpu.sync_copy(srcr.at[s], src_v)
    pltpu.sync_copy(dstr.at[s], dst_v)

    def run(tab, out):
        r0 = jnp.where(s < NS - 1, s * ZB, N - ZB)
        pltpu.sync_copy(tab.at[pl.ds(r0, ZB)], acc.at[pl.ds(r0, ZB)])
        plsc.subcore_barrier()

        def body(j, carry):
            pltpu.async_copy(tab.at[src_v.at[j]], rows_v, sem).wait()
            pltpu.sync_copy(rows_v, acc.at[dst_v.at[j]], add=True)
            return carry

        lax.fori_loop(0, NB, body, 0)
        plsc.subcore_barrier()
        pltpu.sync_copy(acc.at[pl.ds(r0, ZB)], out.at[pl.ds(r0, ZB)])

    @pl.when(c == 0)
    def _():
        run(xsa, outa)

    @pl.when(c == 1)
    def _():
        run(xsb, outb)


def _mm_body(sa_ref, sb_ref, db_ref, wmu_ref, wls_ref, bmu_ref, bls_ref,
             mu_ref, ls_ref):
    db = db_ref[...]
    p0 = sa_ref[...] * db
    p1 = sb_ref[...] * db
    wmu = wmu_ref[...]
    wls = wls_ref[...]
    mu_ref[...] = (
        jnp.dot(p0, wmu[:H, :], preferred_element_type=jnp.float32)
        + jnp.dot(p1, wmu[H:, :], preferred_element_type=jnp.float32)
        + bmu_ref[...]
    )
    ls_ref[...] = (
        jnp.dot(p0, wls[:H, :], preferred_element_type=jnp.float32)
        + jnp.dot(p1, wls[H:, :], preferred_element_type=jnp.float32)
        + bls_ref[...]
    )


_mm = pl.pallas_call(
    _mm_body,
    grid=(N // R,),
    in_specs=[
        pl.BlockSpec((R, H), lambda i: (i, 0)),
        pl.BlockSpec((R, H), lambda i: (i, 0)),
        pl.BlockSpec((R, H), lambda i: (i, 0)),
        pl.BlockSpec((D, D), lambda i: (0, 0)),
        pl.BlockSpec((D, D), lambda i: (0, 0)),
        pl.BlockSpec((1, D), lambda i: (0, 0)),
        pl.BlockSpec((1, D), lambda i: (0, 0)),
    ],
    out_specs=[
        pl.BlockSpec((R, D), lambda i: (i, 0)),
        pl.BlockSpec((R, D), lambda i: (i, 0)),
    ],
    out_shape=[
        jax.ShapeDtypeStruct((N, D), jnp.float32),
        jax.ShapeDtypeStruct((N, D), jnp.float32),
    ],
)


def kernel(x, edge_index, W_mu, b_mu, W_logstd, b_logstd):
    src = edge_index[0].reshape(NS, NB, B)
    dst = edge_index[1].reshape(NS, NB, B)
    ones_up = jnp.ones((B,), jnp.float32)

    degw = _deg_kernel(dst, ones_up)
    degc = degw.reshape(N, 1)

    xsa, xsb, db = _scale(degc, x)
    sa, sb = _agg_kernel(xsa, xsb, src, dst)
    mu, logstd = _mm(sa, sb, db, W_mu, W_logstd,
                     b_mu.reshape(1, D), b_logstd.reshape(1, D))
    return (mu, logstd)


# speedup vs baseline: 34.6232x; 1.6495x over previous
import functools

import jax
import jax.numpy as jnp
from jax import lax
from jax.experimental import pallas as pl
from jax.experimental.pallas import tpu as pltpu
from jax.experimental.pallas import tpu_sc as plsc

N = 10000
E = 160000
D = 256
H = 128
NC = 2
NS = 16
B = 80
NBP = 128
PAD_ROWS = 64
PN = N + PAD_ROWS
EP = NS * NBP * B
G = 8
GB2 = 2 * G
NBODY = NBP // GB2
NBUF = 4
DEPTH = 3
ZB = 640
R = 1000

_mesh = plsc.VectorSubcoreMesh(core_axis_name="c", subcore_axis_name="s")


@functools.partial(
    pl.kernel,
    out_type=jax.ShapeDtypeStruct((PN,), jnp.float32),
    mesh=_mesh,
    scratch_types=[
        pltpu.VMEM((NBP, B), jnp.int32),
        pltpu.VMEM((B,), jnp.float32),
        pltpu.VMEM((ZB,), jnp.float32),
        pltpu.VMEM_SHARED((PN,), jnp.float32),
    ],
)
def _deg_kernel(dstr, ones_hbm, out, dst_v, ones_v, slab_v, dacc):
    c = lax.axis_index("c")
    s = lax.axis_index("s")

    @pl.when(c == 0)
    def _():
        r0 = jnp.where(s < NS - 1, s * ZB, PN - ZB)
        for i in range(ZB // 16):
            slab_v[pl.ds(16 * i, 16)] = jnp.zeros((16,), jnp.float32)
        pltpu.sync_copy(slab_v, dacc.at[pl.ds(r0, ZB)])
        pltpu.sync_copy(ones_hbm, ones_v)
        pltpu.sync_copy(dstr.at[s], dst_v)
        plsc.subcore_barrier()

        def body(j, carry):
            pltpu.sync_copy(ones_v, dacc.at[dst_v.at[j]], add=True)
            return carry

        lax.fori_loop(0, NBP, body, 0)
        plsc.subcore_barrier()
        pltpu.sync_copy(dacc.at[pl.ds(r0, ZB)], slab_v)
        pltpu.sync_copy(slab_v, out.at[pl.ds(r0, ZB)])


def _scale_body(deg_ref, x_ref, xsa_ref, xsb_ref, db_ref):
    dinv = lax.rsqrt(deg_ref[...] + 1.0)
    xs = x_ref[...] * dinv
    xsa_ref[...] = xs[:, :H]
    xsb_ref[...] = xs[:, H:]
    db_ref[...] = jnp.broadcast_to(dinv, (R, H))


_scale = pl.pallas_call(
    _scale_body,
    grid=(N // R,),
    in_specs=[
        pl.BlockSpec((R, 1), lambda i: (i, 0)),
        pl.BlockSpec((R, D), lambda i: (i, 0)),
    ],
    out_specs=[
        pl.BlockSpec((R, H), lambda i: (i, 0)),
        pl.BlockSpec((R, H), lambda i: (i, 0)),
        pl.BlockSpec((R, H), lambda i: (i, 0)),
    ],
    out_shape=[
        jax.ShapeDtypeStruct((PN, H), jnp.float32),
        jax.ShapeDtypeStruct((PN, H), jnp.float32),
        jax.ShapeDtypeStruct((N, H), jnp.float32),
    ],
)


@functools.partial(
    pl.kernel,
    out_type=(
        jax.ShapeDtypeStruct((PN, H), jnp.float32),
        jax.ShapeDtypeStruct((PN, H), jnp.float32),
    ),
    mesh=_mesh,
    scratch_types=[
        pltpu.VMEM((2, G, B), jnp.int32),
        pltpu.VMEM((2, G, B), jnp.int32),
        pltpu.VMEM((NBUF, B, H), jnp.float32),
        pltpu.VMEM_SHARED((PN, H), jnp.float32),
        pltpu.SemaphoreType.DMA,
        pltpu.SemaphoreType.DMA,
    ],
)
def _agg_kernel(xsa, xsb, srcr, dstr, outa, outb, src_v, dst_v, rows_v, acc,
                gsem, isem):
    c = lax.axis_index("c")
    s = lax.axis_index("s")

    def run(tab, out):
        r0 = jnp.where(s < NS - 1, s * ZB, PN - ZB)
        pltpu.sync_copy(tab.at[pl.ds(r0, ZB)], acc.at[pl.ds(r0, ZB)])
        plsc.subcore_barrier()

        def idx_load(slot, g):
            off = pl.multiple_of(g * G, G)
            pltpu.async_copy(srcr.at[s, pl.ds(off, G)], src_v.at[slot], isem)
            pltpu.async_copy(dstr.at[s, pl.ds(off, G)], dst_v.at[slot], isem)

        def idx_wait(slot):
            pltpu.make_async_copy(
                srcr.at[s, pl.ds(0, G)], src_v.at[slot], isem).wait()
            pltpu.make_async_copy(
                dstr.at[s, pl.ds(0, G)], dst_v.at[slot], isem).wait()

        def gather(slot, row, buf):
            pltpu.async_copy(tab.at[src_v.at[slot, row]], rows_v.at[buf], gsem)

        def gather_wait(buf):
            pltpu.make_async_copy(
                tab.at[pl.ds(0, B)], rows_v.at[buf], gsem).wait()

        pltpu.sync_copy(srcr.at[s, pl.ds(0, G)], src_v.at[0])
        pltpu.sync_copy(dstr.at[s, pl.ds(0, G)], dst_v.at[0])
        for b in range(DEPTH):
            gather(0, b, b)

        def body(i, carry):
            idx_load(1, 2 * i + 1)
            for t in range(GB2):
                gather_wait(t % NBUF)
                nt = t + DEPTH
                if nt < GB2:
                    slot, row = (0, nt) if nt < G else (1, nt - G)
                    if nt == G:
                        idx_wait(1)
                    gather(slot, row, nt % NBUF)
                else:
                    @pl.when(i < NBODY - 1)
                    def _(nt=nt):
                        if nt == GB2:
                            idx_wait(0)
                        gather(0, nt - GB2, nt % NBUF)
                if t == G:
                    @pl.when(i < NBODY - 1)
                    def _():
                        idx_load(0, 2 * i + 2)
                sl, rw = (0, t) if t < G else (1, t - G)
                pltpu.sync_copy(
                    rows_v.at[t % NBUF], acc.at[dst_v.at[sl, rw]], add=True)
            return carry

        lax.fori_loop(0, NBODY, body, 0)
        plsc.subcore_barrier()
        pltpu.sync_copy(acc.at[pl.ds(r0, ZB)], out.at[pl.ds(r0, ZB)])

    @pl.when(c == 0)
    def _():
        run(xsa, outa)

    @pl.when(c == 1)
    def _():
        run(xsb, outb)


def _mm_body(sa_ref, sb_ref, db_ref, wmu_ref, wls_ref, bmu_ref, bls_ref,
             mu_ref, ls_ref):
    db = db_ref[...]
    p0 = sa_ref[...] * db
    p1 = sb_ref[...] * db
    wmu = wmu_ref[...]
    wls = wls_ref[...]
    mu_ref[...] = (
        jnp.dot(p0, wmu[:H, :], preferred_element_type=jnp.float32)
        + jnp.dot(p1, wmu[H:, :], preferred_element_type=jnp.float32)
        + bmu_ref[...]
    )
    ls_ref[...] = (
        jnp.dot(p0, wls[:H, :], preferred_element_type=jnp.float32)
        + jnp.dot(p1, wls[H:, :], preferred_element_type=jnp.float32)
        + bls_ref[...]
    )


_mm = pl.pallas_call(
    _mm_body,
    grid=(N // R,),
    in_specs=[
        pl.BlockSpec((R, H), lambda i: (i, 0)),
        pl.BlockSpec((R, H), lambda i: (i, 0)),
        pl.BlockSpec((R, H), lambda i: (i, 0)),
        pl.BlockSpec((D, D), lambda i: (0, 0)),
        pl.BlockSpec((D, D), lambda i: (0, 0)),
        pl.BlockSpec((1, D), lambda i: (0, 0)),
        pl.BlockSpec((1, D), lambda i: (0, 0)),
    ],
    out_specs=[
        pl.BlockSpec((R, D), lambda i: (i, 0)),
        pl.BlockSpec((R, D), lambda i: (i, 0)),
    ],
    out_shape=[
        jax.ShapeDtypeStruct((N, D), jnp.float32),
        jax.ShapeDtypeStruct((N, D), jnp.float32),
    ],
)


def kernel(x, edge_index, W_mu, b_mu, W_logstd, b_logstd):
    pad_idx = N + (jnp.arange(EP - E, dtype=jnp.int32) % PAD_ROWS)
    src = jnp.concatenate([edge_index[0], pad_idx]).reshape(NS, NBP, B)
    dst = jnp.concatenate([edge_index[1], pad_idx]).reshape(NS, NBP, B)
    ones_up = jnp.ones((B,), jnp.float32)

    degw = _deg_kernel(dst, ones_up)
    degc = degw[:N].reshape(N, 1)

    xsa, xsb, db = _scale(degc, x)
    sa, sb = _agg_kernel(xsa, xsb, src, dst)
    mu, logstd = _mm(sa, sb, db, W_mu, W_logstd,
                     b_mu.reshape(1, D), b_logstd.reshape(1, D))
    return (mu, logstd)


# speedup vs baseline: 36.4537x; 1.0529x over previous
import functools

import jax
import jax.numpy as jnp
from jax import lax
from jax.experimental import pallas as pl
from jax.experimental.pallas import tpu as pltpu
from jax.experimental.pallas import tpu_sc as plsc

N = 10000
E = 160000
D = 256
H = 128
NC = 2
NS = 16
B = 80
NBP = 128
PAD_ROWS = 64
PN = N + PAD_ROWS
EP = NS * NBP * B
G = 8
GB2 = 2 * G
NBODY = NBP // GB2
NBUF = 4
DEPTH = 3
ZB = 640
R = 1000

_mesh = plsc.VectorSubcoreMesh(core_axis_name="c", subcore_axis_name="s")


@functools.partial(
    pl.kernel,
    out_type=jax.ShapeDtypeStruct((PN,), jnp.float32),
    mesh=_mesh,
    scratch_types=[
        pltpu.VMEM((NBP, B), jnp.int32),
        pltpu.VMEM((B,), jnp.float32),
        pltpu.VMEM((ZB,), jnp.float32),
        pltpu.VMEM_SHARED((PN,), jnp.float32),
        pltpu.SemaphoreType.DMA,
    ],
)
def _deg_kernel(dstr, ones_hbm, out, dst_v, ones_v, slab_v, dacc, dsem):
    c = lax.axis_index("c")
    s = lax.axis_index("s")

    @pl.when(c == 0)
    def _():
        r0 = jnp.where(s < NS - 1, s * ZB, PN - ZB)
        for i in range(ZB // 16):
            slab_v[pl.ds(16 * i, 16)] = jnp.zeros((16,), jnp.float32)
        pltpu.sync_copy(slab_v, dacc.at[pl.ds(r0, ZB)])
        pltpu.sync_copy(ones_hbm, ones_v)
        pltpu.sync_copy(dstr.at[s], dst_v)
        plsc.subcore_barrier()

        def body(j, carry):
            pltpu.async_copy(ones_v, dacc.at[dst_v.at[j]], dsem, add=True)
            return carry

        lax.fori_loop(0, NBP, body, 0)

        def drain(j, carry):
            pltpu.make_async_copy(ones_v, dacc.at[pl.ds(0, B)], dsem).wait()
            return carry

        lax.fori_loop(0, NBP, drain, 0)
        plsc.subcore_barrier()
        pltpu.sync_copy(dacc.at[pl.ds(r0, ZB)], slab_v)
        pltpu.sync_copy(slab_v, out.at[pl.ds(r0, ZB)])


def _scale_body(deg_ref, x_ref, xsa_ref, xsb_ref, db_ref):
    dinv = lax.rsqrt(deg_ref[...] + 1.0)
    xs = x_ref[...] * dinv
    xsa_ref[...] = xs[:, :H]
    xsb_ref[...] = xs[:, H:]
    db_ref[...] = jnp.broadcast_to(dinv, (R, H))


_scale = pl.pallas_call(
    _scale_body,
    grid=(N // R,),
    in_specs=[
        pl.BlockSpec((R, 1), lambda i: (i, 0)),
        pl.BlockSpec((R, D), lambda i: (i, 0)),
    ],
    out_specs=[
        pl.BlockSpec((R, H), lambda i: (i, 0)),
        pl.BlockSpec((R, H), lambda i: (i, 0)),
        pl.BlockSpec((R, H), lambda i: (i, 0)),
    ],
    out_shape=[
        jax.ShapeDtypeStruct((PN, H), jnp.float32),
        jax.ShapeDtypeStruct((PN, H), jnp.float32),
        jax.ShapeDtypeStruct((N, H), jnp.float32),
    ],
)


@functools.partial(
    pl.kernel,
    out_type=(
        jax.ShapeDtypeStruct((PN, H), jnp.float32),
        jax.ShapeDtypeStruct((PN, H), jnp.float32),
    ),
    mesh=_mesh,
    scratch_types=[
        pltpu.VMEM((2, G, B), jnp.int32),
        pltpu.VMEM((2, G, B), jnp.int32),
        pltpu.VMEM((NBUF, B, H), jnp.float32),
        pltpu.VMEM_SHARED((PN, H), jnp.float32),
        pltpu.SemaphoreType.DMA,
        pltpu.SemaphoreType.DMA,
        pltpu.SemaphoreType.DMA,
    ],
)
def _agg_kernel(xsa, xsb, srcr, dstr, outa, outb, src_v, dst_v, rows_v, acc,
                gsem, isem, ssem):
    c = lax.axis_index("c")
    s = lax.axis_index("s")

    def run(tab, out):
        r0 = jnp.where(s < NS - 1, s * ZB, PN - ZB)
        pltpu.sync_copy(tab.at[pl.ds(r0, ZB)], acc.at[pl.ds(r0, ZB)])
        plsc.subcore_barrier()

        def idx_load(slot, g):
            off = pl.multiple_of(g * G, G)
            pltpu.async_copy(srcr.at[s, pl.ds(off, G)], src_v.at[slot], isem)
            pltpu.async_copy(dstr.at[s, pl.ds(off, G)], dst_v.at[slot], isem)

        def idx_wait(slot):
            pltpu.make_async_copy(
                srcr.at[s, pl.ds(0, G)], src_v.at[slot], isem).wait()
            pltpu.make_async_copy(
                dstr.at[s, pl.ds(0, G)], dst_v.at[slot], isem).wait()

        def gather(slot, row, buf):
            pltpu.async_copy(tab.at[src_v.at[slot, row]], rows_v.at[buf], gsem)

        def gather_wait(buf):
            pltpu.make_async_copy(
                tab.at[pl.ds(0, B)], rows_v.at[buf], gsem).wait()

        def scatter_wait(buf):
            pltpu.make_async_copy(
                rows_v.at[buf], acc.at[pl.ds(0, B)], ssem).wait()

        pltpu.sync_copy(srcr.at[s, pl.ds(0, G)], src_v.at[0])
        pltpu.sync_copy(dstr.at[s, pl.ds(0, G)], dst_v.at[0])
        for b in range(DEPTH):
            gather(0, b, b)

        def body(i, carry):
            idx_load(1, 2 * i + 1)
            for t in range(GB2):
                gather_wait(t % NBUF)
                if t == 0:
                    @pl.when(i > 0)
                    def _():
                        scatter_wait((t + 3) % NBUF)
                else:
                    scatter_wait((t + 3) % NBUF)
                nt = t + DEPTH
                if nt < GB2:
                    slot, row = (0, nt) if nt < G else (1, nt - G)
                    if nt == G:
                        idx_wait(1)
                    gather(slot, row, nt % NBUF)
                else:
                    @pl.when(i < NBODY - 1)
                    def _(nt=nt):
                        if nt == GB2:
                            idx_wait(0)
                        gather(0, nt - GB2, nt % NBUF)
                if t == G:
                    @pl.when(i < NBODY - 1)
                    def _():
                        idx_load(0, 2 * i + 2)
                sl, rw = (0, t) if t < G else (1, t - G)
                pltpu.async_copy(
                    rows_v.at[t % NBUF], acc.at[dst_v.at[sl, rw]], ssem,
                    add=True)
            return carry

        lax.fori_loop(0, NBODY, body, 0)
        scatter_wait((NBP - 1) % NBUF)
        plsc.subcore_barrier()
        pltpu.sync_copy(acc.at[pl.ds(r0, ZB)], out.at[pl.ds(r0, ZB)])

    @pl.when(c == 0)
    def _():
        run(xsa, outa)

    @pl.when(c == 1)
    def _():
        run(xsb, outb)


def _mm_body(sa_ref, sb_ref, db_ref, wmu_ref, wls_ref, bmu_ref, bls_ref,
             mu_ref, ls_ref):
    db = db_ref[...]
    p0 = sa_ref[...] * db
    p1 = sb_ref[...] * db
    wmu = wmu_ref[...]
    wls = wls_ref[...]
    mu_ref[...] = (
        jnp.dot(p0, wmu[:H, :], preferred_element_type=jnp.float32)
        + jnp.dot(p1, wmu[H:, :], preferred_element_type=jnp.float32)
        + bmu_ref[...]
    )
    ls_ref[...] = (
        jnp.dot(p0, wls[:H, :], preferred_element_type=jnp.float32)
        + jnp.dot(p1, wls[H:, :], preferred_element_type=jnp.float32)
        + bls_ref[...]
    )


_mm = pl.pallas_call(
    _mm_body,
    grid=(N // R,),
    in_specs=[
        pl.BlockSpec((R, H), lambda i: (i, 0)),
        pl.BlockSpec((R, H), lambda i: (i, 0)),
        pl.BlockSpec((R, H), lambda i: (i, 0)),
        pl.BlockSpec((D, D), lambda i: (0, 0)),
        pl.BlockSpec((D, D), lambda i: (0, 0)),
        pl.BlockSpec((1, D), lambda i: (0, 0)),
        pl.BlockSpec((1, D), lambda i: (0, 0)),
    ],
    out_specs=[
        pl.BlockSpec((R, D), lambda i: (i, 0)),
        pl.BlockSpec((R, D), lambda i: (i, 0)),
    ],
    out_shape=[
        jax.ShapeDtypeStruct((N, D), jnp.float32),
        jax.ShapeDtypeStruct((N, D), jnp.float32),
    ],
)


def kernel(x, edge_index, W_mu, b_mu, W_logstd, b_logstd):
    pad_idx = N + (jnp.arange(EP - E, dtype=jnp.int32) % PAD_ROWS)
    src = jnp.concatenate([edge_index[0], pad_idx]).reshape(NS, NBP, B)
    dst = jnp.concatenate([edge_index[1], pad_idx]).reshape(NS, NBP, B)
    ones_up = jnp.ones((B,), jnp.float32)

    degw = _deg_kernel(dst, ones_up)
    degc = degw[:N].reshape(N, 1)

    xsa, xsb, db = _scale(degc, x)
    sa, sb = _agg_kernel(xsa, xsb, src, dst)
    mu, logstd = _mm(sa, sb, db, W_mu, W_logstd,
                     b_mu.reshape(1, D), b_logstd.reshape(1, D))
    return (mu, logstd)
